# Initial kernel scaffold; baseline (speedup 1.0000x reference)
import jax
import jax.numpy as jnp
from jax.experimental import pallas as pl


def kernel(x, edge_index, W_lin, b_lin, W_ain, b_ain, W_aout, b_aout, ln1_g, ln1_b, W_mlp, b_mlp, ln2_g, ln2_b):
    raise NotImplementedError("write your pallas kernel here")


# speedup vs baseline: 1.9525x; 1.9525x over previous
import functools

import jax
import jax.numpy as jnp
from jax import lax
from jax.experimental import pallas as pl
from jax.experimental.pallas import tpu as pltpu
from jax.experimental.pallas import tpu_sc as plsc

LIST_DIM = 4
N_NODES = 10000
N_PAD = 10240
N_EDGES = 320000
D = 128

N_TILES = 16
EDGES_PER_TILE = N_EDGES // N_TILES
BATCH = 128
CHUNK = 32
N_BATCH = 160
N_CHUNK = N_BATCH // CHUNK
EDGES_PAD = N_BATCH * BATCH
ROWS_PER_TILE = N_PAD // N_TILES
DUMMY_ROW = N_NODES


def _sc_agg_body(xflat, gidx, sidx, zeros, agg, acc, gidx_v, sidx_v,
                 rows, sem):
    c = lax.axis_index("c")
    t = lax.axis_index("s")
    row0 = t * ROWS_PER_TILE

    for l in range(LIST_DIM):
        pltpu.sync_copy(zeros.at[pl.ds(row0, ROWS_PER_TILE)],
                        acc.at[pl.ds(row0, ROWS_PER_TILE)])
        plsc.subcore_barrier()

        def chunk_body(k, carry):
            b0 = k * CHUNK
            pltpu.sync_copy(gidx.at[c, l, t, pl.ds(b0, CHUNK)], gidx_v)
            pltpu.sync_copy(sidx.at[c, t, pl.ds(b0, CHUNK)], sidx_v)

            pltpu.async_copy(xflat.at[gidx_v.at[0]], rows.at[pl.ds(0, BATCH)], sem)
            pltpu.async_copy(xflat.at[gidx_v.at[1]], rows.at[pl.ds(BATCH, BATCH)], sem)

            def body(b, carry2):
                slot = (b % 2) * BATCH
                buf = rows.at[pl.ds(slot, BATCH)]
                pltpu.make_async_copy(xflat.at[gidx_v.at[b]], buf, sem).wait()
                pltpu.sync_copy(buf, acc.at[sidx_v.at[b]], add=True)

                @pl.when(b + 2 < CHUNK)
                def _():
                    pltpu.async_copy(xflat.at[gidx_v.at[b + 2]], buf, sem)

                return carry2

            lax.fori_loop(0, CHUNK, body, 0)
            return carry

        lax.fori_loop(0, N_CHUNK, chunk_body, 0)
        plsc.subcore_barrier()

        pltpu.sync_copy(acc.at[pl.ds(row0, ROWS_PER_TILE)],
                        agg.at[c, l, pl.ds(row0, ROWS_PER_TILE)])
        plsc.subcore_barrier()


def _make_sc_agg():
    mesh = plsc.VectorSubcoreMesh(core_axis_name="c", subcore_axis_name="s")
    return functools.partial(
        pl.kernel,
        out_type=jax.ShapeDtypeStruct((2, LIST_DIM, N_PAD, D), jnp.float32),
        mesh=mesh,
        scratch_types=[
            pltpu.VMEM_SHARED((N_PAD, D), jnp.float32),
            pltpu.VMEM((CHUNK, BATCH), jnp.int32),
            pltpu.VMEM((CHUNK, BATCH), jnp.int32),
            pltpu.VMEM((2 * BATCH, D), jnp.float32),
            pltpu.SemaphoreType.DMA,
        ],
    )(_sc_agg_body)


def _tc_body(x_ref, in_ref, og_ref, wl_ref, bl_ref, wa_ref, ba_ref,
             wo_ref, bo_ref, g1_ref, c1_ref, wm_ref, bm_ref, g2_ref, c2_ref,
             o_ref):
    f32 = jnp.float32
    p = jnp.dot(x_ref[0], wl_ref[...], preferred_element_type=f32) + bl_ref[...]
    ai = jnp.dot(in_ref[0], wa_ref[...], preferred_element_type=f32) + ba_ref[...]
    ao = jnp.dot(og_ref[0], wo_ref[...], preferred_element_type=f32) + bo_ref[...]
    o = jnp.concatenate([p, ai, ao], axis=-1)
    o = o * jax.nn.sigmoid(o)
    m = jnp.mean(o, axis=-1, keepdims=True)
    v = jnp.mean((o - m) ** 2, axis=-1, keepdims=True)
    o = (o - m) * lax.rsqrt(v + 1e-5) * g1_ref[...] + c1_ref[...]
    h = jnp.dot(o, wm_ref[...], preferred_element_type=f32) + bm_ref[...]
    h = h * jax.nn.sigmoid(h)
    m2 = jnp.mean(h, axis=-1, keepdims=True)
    v2 = jnp.mean((h - m2) ** 2, axis=-1, keepdims=True)
    o_ref[0] = (h - m2) * lax.rsqrt(v2 + 1e-5) * g2_ref[...] + c2_ref[...]


def _make_tc_dense(block):
    grid = (LIST_DIM, N_NODES // block)
    row_spec = pl.BlockSpec((1, block, D), lambda l, i: (l, i, 0))
    full = lambda shape: pl.BlockSpec(shape, lambda l, i: (0,) * len(shape))
    return pl.pallas_call(
        _tc_body,
        grid=grid,
        in_specs=[
            row_spec, row_spec, row_spec,
            full((D, D)), full((1, D)),
            full((D, D)), full((1, D)),
            full((D, D)), full((1, D)),
            full((1, 3 * D)), full((1, 3 * D)),
            full((3 * D, D)), full((1, D)),
            full((1, D)), full((1, D)),
        ],
        out_specs=row_spec,
        out_shape=jax.ShapeDtypeStruct((LIST_DIM, N_NODES, D), jnp.float32),
    )


@jax.jit
def kernel(x, edge_index, W_lin, b_lin, W_ain, b_ain, W_aout, b_aout,
           ln1_g, ln1_b, W_mlp, b_mlp, ln2_g, ln2_b):
    xflat = x.reshape(LIST_DIM * N_NODES, D)

    ei = edge_index.astype(jnp.int32)
    dst, src = ei[0], ei[1]
    shift = (jnp.arange(LIST_DIM, dtype=jnp.int32) * N_NODES)[None, :, None, None]
    n_dummy = EDGES_PAD - EDGES_PER_TILE
    g = jnp.stack([src, dst]).reshape(2, 1, N_TILES, EDGES_PER_TILE) + shift
    gpad = jnp.zeros((2, LIST_DIM, N_TILES, n_dummy), jnp.int32)
    gidx = jnp.concatenate([g, gpad], axis=-1)
    gidx = gidx.reshape(2, LIST_DIM, N_TILES, N_BATCH, BATCH)
    s = jnp.stack([dst, src]).reshape(2, N_TILES, EDGES_PER_TILE)
    spad = jnp.full((2, N_TILES, n_dummy), DUMMY_ROW, jnp.int32)
    sidx = jnp.concatenate([s, spad], axis=-1).reshape(2, N_TILES, N_BATCH, BATCH)
    zeros = jnp.zeros((N_PAD, D), jnp.float32)

    agg = _make_sc_agg()(xflat, gidx, sidx, zeros)

    out = _make_tc_dense(1000)(
        x, agg[0], agg[1],
        W_lin.T, b_lin[None, :], W_ain.T, b_ain[None, :],
        W_aout.T, b_aout[None, :], ln1_g[None, :], ln1_b[None, :],
        W_mlp.T, b_mlp[None, :], ln2_g[None, :], ln2_b[None, :],
    )
    return out
